# Initial kernel scaffold; baseline (speedup 1.0000x reference)
import jax
import jax.numpy as jnp
from jax.experimental import pallas as pl


def kernel(x, adj, W1, b1, Wmu, bmu, Ws, bs):
    raise NotImplementedError("write your pallas kernel here")


# speedup vs baseline: 8.6342x; 8.6342x over previous
import functools

import jax
import jax.numpy as jnp
from jax import lax
from jax.experimental import pallas as pl
from jax.experimental.pallas import tpu as pltpu
from jax.experimental.pallas import tpu_sc as plsc

N = 10000
E = 320000
IN = 128
HID2 = 256
OUT = 64

NC = 2
NS = 16
NP = 10240
RPT = NP // NS
DUMP = 10200
B = 128

EPAD_DEG = ((E + NC * NS * B - 1) // (NC * NS * B)) * (NC * NS * B)
EPAD_PROP = ((E + NS * B - 1) // (NS * B)) * (NS * B)
EPAD = max(EPAD_DEG, EPAD_PROP)


def _sc_mesh():
    return plsc.VectorSubcoreMesh(
        core_axis_name="c", subcore_axis_name="s", num_cores=NC, num_subcores=NS
    )


def _prop_body(colsplit, uh, adj, zeros_hbm, out, src_v, dst_v, rows_v, acc):
    c = lax.axis_index("c")
    s = lax.axis_index("s")
    if colsplit:
        ept = EPAD // NS
        ebase = s * ept
    else:
        ept = EPAD // (NS * NC)
        ebase = (s * NC + c) * ept

    pltpu.sync_copy(zeros_hbm, acc.at[pl.ds(s * RPT, RPT)])
    plsc.subcore_barrier()

    def step(k, carry):
        base = ebase + k * B
        pltpu.sync_copy(adj.at[0, pl.ds(base, B)], src_v)
        pltpu.sync_copy(adj.at[1, pl.ds(base, B)], dst_v)
        if colsplit:
            pltpu.sync_copy(uh.at[c].at[src_v], rows_v)
        else:
            pltpu.sync_copy(uh.at[src_v], rows_v)
        pltpu.sync_copy(rows_v, acc.at[dst_v], add=True)
        return carry

    lax.fori_loop(0, ept // B, step, 0)
    plsc.subcore_barrier()
    pltpu.sync_copy(acc.at[pl.ds(s * RPT, RPT)], out.at[c, pl.ds(s * RPT, RPT)])


def _prop_call(colsplit, uh, adj_pad, zeros_c):
    return pl.kernel(
        functools.partial(_prop_body, colsplit),
        out_type=jax.ShapeDtypeStruct((NC, NP, IN), jnp.float32),
        mesh=_sc_mesh(),
        scratch_types=[
            pltpu.VMEM((B,), jnp.int32),
            pltpu.VMEM((B,), jnp.int32),
            pltpu.VMEM((B, IN), jnp.float32),
            pltpu.VMEM_SHARED((NP, IN), jnp.float32),
        ],
    )(uh, adj_pad, zeros_c)


BR = 1000


def _dinv_of(degp_blk):
    deg = degp_blk[0, :, 0:1] + degp_blk[1, :, 0:1] + 1.0
    return lax.rsqrt(deg)


def _z1u1_body(x_ref, w1_ref, degp_ref, u1_ref):
    z = jnp.dot(x_ref[...], w1_ref[...], preferred_element_type=jnp.float32)
    u = z * _dinv_of(degp_ref[...])
    u1_ref[0, :, :] = u[:, :IN]
    u1_ref[1, :, :] = u[:, IN:]


def _z1u1_call(x, W1, degp):
    return pl.pallas_call(
        _z1u1_body,
        grid=(N // BR,),
        in_specs=[
            pl.BlockSpec((BR, IN), lambda i: (i, 0)),
            pl.BlockSpec((IN, HID2), lambda i: (0, 0)),
            pl.BlockSpec((NC, BR, IN), lambda i: (0, i, 0)),
        ],
        out_specs=pl.BlockSpec((NC, BR, IN), lambda i: (0, i, 0)),
        out_shape=jax.ShapeDtypeStruct((NC, N, IN), jnp.float32),
    )(x, W1, degp)


def _hz2_body(p1_ref, u1_ref, degp_ref, b1_ref, wc_ref, u2_ref):
    dinv = _dinv_of(degp_ref[...])
    pre = jnp.concatenate(
        [p1_ref[0, :, :] + u1_ref[0, :, :], p1_ref[1, :, :] + u1_ref[1, :, :]],
        axis=1,
    )
    h = jnp.maximum(pre * dinv + b1_ref[...], 0.0)
    z2 = jnp.dot(h, wc_ref[...], preferred_element_type=jnp.float32)
    u2_ref[...] = z2 * dinv


def _hz2_call(P1, u1, degp, b1, Wc):
    return pl.pallas_call(
        _hz2_body,
        grid=(N // BR,),
        in_specs=[
            pl.BlockSpec((NC, BR, IN), lambda i: (0, i, 0)),
            pl.BlockSpec((NC, BR, IN), lambda i: (0, i, 0)),
            pl.BlockSpec((NC, BR, IN), lambda i: (0, i, 0)),
            pl.BlockSpec((1, HID2), lambda i: (0, 0)),
            pl.BlockSpec((HID2, 2 * OUT), lambda i: (0, 0)),
        ],
        out_specs=pl.BlockSpec((BR, 2 * OUT), lambda i: (i, 0)),
        out_shape=jax.ShapeDtypeStruct((N, 2 * OUT), jnp.float32),
    )(P1, u1, degp, b1, Wc)


def _out_body(p2_ref, u2_ref, degp_ref, bc_ref, o_ref):
    dinv = _dinv_of(degp_ref[...])
    pre = p2_ref[0, :, :] + p2_ref[1, :, :] + u2_ref[...]
    o_ref[...] = jnp.maximum(pre * dinv + bc_ref[...], 0.0)


def _out_call(P2, u2, degp, bc):
    return pl.pallas_call(
        _out_body,
        grid=(N // BR,),
        in_specs=[
            pl.BlockSpec((NC, BR, 2 * OUT), lambda i: (0, i, 0)),
            pl.BlockSpec((BR, 2 * OUT), lambda i: (i, 0)),
            pl.BlockSpec((NC, BR, IN), lambda i: (0, i, 0)),
            pl.BlockSpec((1, 2 * OUT), lambda i: (0, 0)),
        ],
        out_specs=pl.BlockSpec((BR, 2 * OUT), lambda i: (i, 0)),
        out_shape=jax.ShapeDtypeStruct((N, 2 * OUT), jnp.float32),
    )(P2, u2, degp, bc)


def kernel(x, adj, W1, b1, Wmu, bmu, Ws, bs):
    pad = EPAD - E
    pad_edges = jnp.stack(
        [jnp.zeros((pad,), jnp.int32), jnp.full((pad,), DUMP, jnp.int32)]
    )
    adj_pad = jnp.concatenate([adj, pad_edges], axis=1)

    zeros128 = jnp.zeros((RPT, IN), jnp.float32)
    ones_mat = jnp.ones((N, IN), jnp.float32)
    Wc = jnp.concatenate([Wmu, Ws], axis=1)
    bc = jnp.concatenate([bmu, bs]).reshape(1, 2 * OUT)
    b1r = b1.reshape(1, HID2)

    degp = _prop_call(False, ones_mat, adj_pad, zeros128)
    u1 = _z1u1_call(x, W1, degp)
    P1 = _prop_call(True, u1, adj_pad, zeros128)
    u2 = _hz2_call(P1[:, :N, :], u1, degp, b1r, Wc)
    P2 = _prop_call(False, u2, adj_pad, zeros128)
    o = _out_call(P2[:, :N, :], u2, degp, bc)
    return (o[:, :OUT], o[:, OUT:])


# speedup vs baseline: 9.3376x; 1.0815x over previous
import functools

import jax
import jax.numpy as jnp
from jax import lax
from jax.experimental import pallas as pl
from jax.experimental.pallas import tpu as pltpu
from jax.experimental.pallas import tpu_sc as plsc

N = 10000
E = 320000
IN = 128
HID2 = 256
OUT = 64

NC = 2
NS = 16
NP = 10240
RPT = NP // NS
DUMP = 10200
B = 128

EPAD = ((E + NC * NS * 2 * B - 1) // (NC * NS * 2 * B)) * (NC * NS * 2 * B)


def _sc_mesh():
    return plsc.VectorSubcoreMesh(
        core_axis_name="c", subcore_axis_name="s", num_cores=NC, num_subcores=NS
    )


KB = 16


def _deg_body(adj, ones_hbm, zeros_hbm, out, d0, d1, ones_v, acc, i0, i1, ss):
    c = lax.axis_index("c")
    s = lax.axis_index("s")
    ept = EPAD // (NS * NC)
    ebase = (s * NC + c) * ept

    pltpu.sync_copy(zeros_hbm, acc.at[pl.ds(s * RPT, RPT)])
    pltpu.sync_copy(ones_hbm, ones_v)
    plsc.subcore_barrier()

    def pair(t, carry):
        a = ebase + t * 2 * B
        b = a + B
        pltpu.async_copy(adj.at[1, pl.ds(a, B)], d0, i0)
        pltpu.async_copy(adj.at[1, pl.ds(b, B)], d1, i1)
        pltpu.make_async_copy(adj.at[1, pl.ds(a, B)], d0, i0).wait()
        pltpu.async_copy(ones_v, acc.at[d0], ss, add=True)
        pltpu.make_async_copy(adj.at[1, pl.ds(b, B)], d1, i1).wait()
        pltpu.make_async_copy(ones_v, acc.at[d0], ss).wait()
        pltpu.async_copy(ones_v, acc.at[d1], ss, add=True)
        pltpu.make_async_copy(ones_v, acc.at[d1], ss).wait()
        return carry

    lax.fori_loop(0, ept // (2 * B), pair, 0)
    plsc.subcore_barrier()
    pltpu.sync_copy(acc.at[pl.ds(s * RPT, RPT)], out.at[c, pl.ds(s * RPT, RPT)])


def _deg_call(adj_pad, ones_b, zeros_c):
    return pl.kernel(
        _deg_body,
        out_type=jax.ShapeDtypeStruct((NC, NP, IN), jnp.float32),
        mesh=_sc_mesh(),
        scratch_types=[
            pltpu.VMEM((B,), jnp.int32),
            pltpu.VMEM((B,), jnp.int32),
            pltpu.VMEM((B, IN), jnp.float32),
            pltpu.VMEM_SHARED((NP, IN), jnp.float32),
            pltpu.SemaphoreType.DMA,
            pltpu.SemaphoreType.DMA,
            pltpu.SemaphoreType.DMA,
        ],
    )(adj_pad, ones_b, zeros_c)


def _prop_body(colsplit, uh, adj, zeros_hbm, out,
               sr0, sr1, d0, d1, r0, r1, acc, i0, i1, g0, g1, ss):
    c = lax.axis_index("c")
    s = lax.axis_index("s")
    if colsplit:
        ept = EPAD // NS
        ebase = s * ept
        srcref = lambda sb: uh.at[c].at[sb]
    else:
        ept = EPAD // (NS * NC)
        ebase = (s * NC + c) * ept
        srcref = lambda sb: uh.at[sb]

    pltpu.sync_copy(zeros_hbm, acc.at[pl.ds(s * RPT, RPT)])
    plsc.subcore_barrier()

    def pair(t, carry):
        a = ebase + t * 2 * B
        b = a + B
        pltpu.async_copy(adj.at[0, pl.ds(a, B)], sr0, i0)
        pltpu.async_copy(adj.at[1, pl.ds(a, B)], d0, i0)
        pltpu.async_copy(adj.at[0, pl.ds(b, B)], sr1, i1)
        pltpu.async_copy(adj.at[1, pl.ds(b, B)], d1, i1)
        pltpu.make_async_copy(adj.at[0, pl.ds(a, B)], sr0, i0).wait()
        pltpu.make_async_copy(adj.at[1, pl.ds(a, B)], d0, i0).wait()
        pltpu.async_copy(srcref(sr0), r0, g0)
        pltpu.make_async_copy(adj.at[0, pl.ds(b, B)], sr1, i1).wait()
        pltpu.make_async_copy(adj.at[1, pl.ds(b, B)], d1, i1).wait()
        pltpu.make_async_copy(srcref(sr0), r0, g0).wait()
        pltpu.async_copy(r0, acc.at[d0], ss, add=True)
        pltpu.async_copy(srcref(sr1), r1, g1)
        pltpu.make_async_copy(srcref(sr1), r1, g1).wait()
        pltpu.make_async_copy(r0, acc.at[d0], ss).wait()
        pltpu.async_copy(r1, acc.at[d1], ss, add=True)
        pltpu.make_async_copy(r1, acc.at[d1], ss).wait()
        return carry

    lax.fori_loop(0, ept // (2 * B), pair, 0)
    plsc.subcore_barrier()
    pltpu.sync_copy(acc.at[pl.ds(s * RPT, RPT)], out.at[c, pl.ds(s * RPT, RPT)])


def _prop_call(colsplit, uh, adj_pad, zeros_c):
    return pl.kernel(
        functools.partial(_prop_body, colsplit),
        out_type=jax.ShapeDtypeStruct((NC, NP, IN), jnp.float32),
        mesh=_sc_mesh(),
        scratch_types=[
            pltpu.VMEM((B,), jnp.int32),
            pltpu.VMEM((B,), jnp.int32),
            pltpu.VMEM((B,), jnp.int32),
            pltpu.VMEM((B,), jnp.int32),
            pltpu.VMEM((B, IN), jnp.float32),
            pltpu.VMEM((B, IN), jnp.float32),
            pltpu.VMEM_SHARED((NP, IN), jnp.float32),
            pltpu.SemaphoreType.DMA,
            pltpu.SemaphoreType.DMA,
            pltpu.SemaphoreType.DMA,
            pltpu.SemaphoreType.DMA,
            pltpu.SemaphoreType.DMA,
        ],
    )(uh, adj_pad, zeros_c)


BR = 1000


def _dinv_of(degp_blk):
    deg = degp_blk[0, :, 0:1] + degp_blk[1, :, 0:1] + 1.0
    return lax.rsqrt(deg)


def _z1u1_body(x_ref, w1_ref, degp_ref, u1_ref):
    z = jnp.dot(x_ref[...], w1_ref[...], preferred_element_type=jnp.float32)
    u = z * _dinv_of(degp_ref[...])
    u1_ref[0, :, :] = u[:, :IN]
    u1_ref[1, :, :] = u[:, IN:]


def _z1u1_call(x, W1, degp):
    return pl.pallas_call(
        _z1u1_body,
        grid=(N // BR,),
        in_specs=[
            pl.BlockSpec((BR, IN), lambda i: (i, 0)),
            pl.BlockSpec((IN, HID2), lambda i: (0, 0)),
            pl.BlockSpec((NC, BR, IN), lambda i: (0, i, 0)),
        ],
        out_specs=pl.BlockSpec((NC, BR, IN), lambda i: (0, i, 0)),
        out_shape=jax.ShapeDtypeStruct((NC, N, IN), jnp.float32),
    )(x, W1, degp)


def _hz2_body(p1_ref, u1_ref, degp_ref, b1_ref, wc_ref, u2_ref):
    dinv = _dinv_of(degp_ref[...])
    pre = jnp.concatenate(
        [p1_ref[0, :, :] + u1_ref[0, :, :], p1_ref[1, :, :] + u1_ref[1, :, :]],
        axis=1,
    )
    h = jnp.maximum(pre * dinv + b1_ref[...], 0.0)
    z2 = jnp.dot(h, wc_ref[...], preferred_element_type=jnp.float32)
    u2_ref[...] = z2 * dinv


def _hz2_call(P1, u1, degp, b1, Wc):
    return pl.pallas_call(
        _hz2_body,
        grid=(N // BR,),
        in_specs=[
            pl.BlockSpec((NC, BR, IN), lambda i: (0, i, 0)),
            pl.BlockSpec((NC, BR, IN), lambda i: (0, i, 0)),
            pl.BlockSpec((NC, BR, IN), lambda i: (0, i, 0)),
            pl.BlockSpec((1, HID2), lambda i: (0, 0)),
            pl.BlockSpec((HID2, 2 * OUT), lambda i: (0, 0)),
        ],
        out_specs=pl.BlockSpec((BR, 2 * OUT), lambda i: (i, 0)),
        out_shape=jax.ShapeDtypeStruct((N, 2 * OUT), jnp.float32),
    )(P1, u1, degp, b1, Wc)


def _out_body(p2_ref, u2_ref, degp_ref, bc_ref, o_ref):
    dinv = _dinv_of(degp_ref[...])
    pre = p2_ref[0, :, :] + p2_ref[1, :, :] + u2_ref[...]
    o_ref[...] = jnp.maximum(pre * dinv + bc_ref[...], 0.0)


def _out_call(P2, u2, degp, bc):
    return pl.pallas_call(
        _out_body,
        grid=(N // BR,),
        in_specs=[
            pl.BlockSpec((NC, BR, 2 * OUT), lambda i: (0, i, 0)),
            pl.BlockSpec((BR, 2 * OUT), lambda i: (i, 0)),
            pl.BlockSpec((NC, BR, IN), lambda i: (0, i, 0)),
            pl.BlockSpec((1, 2 * OUT), lambda i: (0, 0)),
        ],
        out_specs=pl.BlockSpec((BR, 2 * OUT), lambda i: (i, 0)),
        out_shape=jax.ShapeDtypeStruct((N, 2 * OUT), jnp.float32),
    )(P2, u2, degp, bc)


def kernel(x, adj, W1, b1, Wmu, bmu, Ws, bs):
    pad = EPAD - E
    pad_edges = jnp.stack(
        [jnp.zeros((pad,), jnp.int32), jnp.full((pad,), DUMP, jnp.int32)]
    )
    adj_pad = jnp.concatenate([adj, pad_edges], axis=1)

    zeros128 = jnp.zeros((RPT, IN), jnp.float32)
    ones_b = jnp.ones((B, IN), jnp.float32)
    Wc = jnp.concatenate([Wmu, Ws], axis=1)
    bc = jnp.concatenate([bmu, bs]).reshape(1, 2 * OUT)
    b1r = b1.reshape(1, HID2)

    degp = _deg_call(adj_pad, ones_b, zeros128)
    u1 = _z1u1_call(x, W1, degp)
    P1 = _prop_call(True, u1, adj_pad, zeros128)
    u2 = _hz2_call(P1[:, :N, :], u1, degp, b1r, Wc)
    P2 = _prop_call(False, u2, adj_pad, zeros128)
    o = _out_call(P2[:, :N, :], u2, degp, bc)
    return (o[:, :OUT], o[:, OUT:])


# speedup vs baseline: 10.0148x; 1.0725x over previous
import functools

import jax
import jax.numpy as jnp
from jax import lax
from jax.experimental import pallas as pl
from jax.experimental.pallas import tpu as pltpu
from jax.experimental.pallas import tpu_sc as plsc

N = 10000
E = 320000
IN = 128
HID2 = 256
OUT = 64

NC = 2
NS = 16
NP = 10240
RPT = NP // NS
DUMP = 10200
B = 128

EPAD = ((E + NC * NS * 2 * B - 1) // (NC * NS * 2 * B)) * (NC * NS * 2 * B)


def _sc_mesh():
    return plsc.VectorSubcoreMesh(
        core_axis_name="c", subcore_axis_name="s", num_cores=NC, num_subcores=NS
    )


KB = 16


def _deg_body(adj, ones_hbm, zeros_hbm, out, d0, d1, ones_v, acc, i0, i1, ss):
    c = lax.axis_index("c")
    s = lax.axis_index("s")
    ept = EPAD // (NS * NC)
    ebase = (s * NC + c) * ept

    pltpu.sync_copy(zeros_hbm, acc.at[pl.ds(s * RPT, RPT)])
    pltpu.sync_copy(ones_hbm, ones_v)
    plsc.subcore_barrier()

    pltpu.async_copy(adj.at[1, pl.ds(EPAD - B, B)], d1, i1)
    pltpu.make_async_copy(adj.at[1, pl.ds(EPAD - B, B)], d1, i1).wait()
    pltpu.async_copy(ones_v, acc.at[d1], ss, add=True)

    def pair(t, carry):
        a = ebase + t * 2 * B
        b = a + B
        pltpu.async_copy(adj.at[1, pl.ds(a, B)], d0, i0)
        pltpu.make_async_copy(ones_v, acc.at[d1], ss).wait()
        pltpu.async_copy(adj.at[1, pl.ds(b, B)], d1, i1)
        pltpu.make_async_copy(adj.at[1, pl.ds(a, B)], d0, i0).wait()
        pltpu.async_copy(ones_v, acc.at[d0], ss, add=True)
        pltpu.make_async_copy(adj.at[1, pl.ds(b, B)], d1, i1).wait()
        pltpu.make_async_copy(ones_v, acc.at[d0], ss).wait()
        pltpu.async_copy(ones_v, acc.at[d1], ss, add=True)
        return carry

    lax.fori_loop(0, ept // (2 * B), pair, 0)
    pltpu.make_async_copy(ones_v, acc.at[d1], ss).wait()
    plsc.subcore_barrier()
    pltpu.sync_copy(acc.at[pl.ds(s * RPT, RPT)], out.at[c, pl.ds(s * RPT, RPT)])


def _deg_call(adj_pad, ones_b, zeros_c):
    return pl.kernel(
        _deg_body,
        out_type=jax.ShapeDtypeStruct((NC, NP, IN), jnp.float32),
        mesh=_sc_mesh(),
        scratch_types=[
            pltpu.VMEM((B,), jnp.int32),
            pltpu.VMEM((B,), jnp.int32),
            pltpu.VMEM((B, IN), jnp.float32),
            pltpu.VMEM_SHARED((NP, IN), jnp.float32),
            pltpu.SemaphoreType.DMA,
            pltpu.SemaphoreType.DMA,
            pltpu.SemaphoreType.DMA,
        ],
    )(adj_pad, ones_b, zeros_c)


def _prop_body(colsplit, uh, adj, zeros_hbm, out,
               sr0, sr1, d0, d1, r0, r1, acc, i0, i1, g0, g1, ss):
    c = lax.axis_index("c")
    s = lax.axis_index("s")
    if colsplit:
        ept = EPAD // NS
        ebase = s * ept
        srcref = lambda sb: uh.at[c].at[sb]
    else:
        ept = EPAD // (NS * NC)
        ebase = (s * NC + c) * ept
        srcref = lambda sb: uh.at[sb]

    pltpu.sync_copy(zeros_hbm, acc.at[pl.ds(s * RPT, RPT)])
    plsc.subcore_barrier()

    pltpu.async_copy(adj.at[1, pl.ds(EPAD - B, B)], d1, i1)
    pltpu.make_async_copy(adj.at[1, pl.ds(EPAD - B, B)], d1, i1).wait()
    pltpu.async_copy(r1, acc.at[d1], ss, add=True)

    def pair(t, carry):
        a = ebase + t * 2 * B
        b = a + B
        pltpu.async_copy(adj.at[0, pl.ds(a, B)], sr0, i0)
        pltpu.async_copy(adj.at[1, pl.ds(a, B)], d0, i0)
        pltpu.make_async_copy(r1, acc.at[d1], ss).wait()
        pltpu.async_copy(adj.at[0, pl.ds(b, B)], sr1, i1)
        pltpu.async_copy(adj.at[1, pl.ds(b, B)], d1, i1)
        pltpu.make_async_copy(adj.at[0, pl.ds(a, B)], sr0, i0).wait()
        pltpu.make_async_copy(adj.at[1, pl.ds(a, B)], d0, i0).wait()
        pltpu.async_copy(srcref(sr0), r0, g0)
        pltpu.make_async_copy(adj.at[0, pl.ds(b, B)], sr1, i1).wait()
        pltpu.make_async_copy(adj.at[1, pl.ds(b, B)], d1, i1).wait()
        pltpu.async_copy(srcref(sr1), r1, g1)
        pltpu.make_async_copy(srcref(sr0), r0, g0).wait()
        pltpu.async_copy(r0, acc.at[d0], ss, add=True)
        pltpu.make_async_copy(srcref(sr1), r1, g1).wait()
        pltpu.make_async_copy(r0, acc.at[d0], ss).wait()
        pltpu.async_copy(r1, acc.at[d1], ss, add=True)
        return carry

    lax.fori_loop(0, ept // (2 * B), pair, 0)
    pltpu.make_async_copy(r1, acc.at[d1], ss).wait()
    plsc.subcore_barrier()
    pltpu.sync_copy(acc.at[pl.ds(s * RPT, RPT)], out.at[c, pl.ds(s * RPT, RPT)])


def _prop_call(colsplit, uh, adj_pad, zeros_c):
    return pl.kernel(
        functools.partial(_prop_body, colsplit),
        out_type=jax.ShapeDtypeStruct((NC, NP, IN), jnp.float32),
        mesh=_sc_mesh(),
        scratch_types=[
            pltpu.VMEM((B,), jnp.int32),
            pltpu.VMEM((B,), jnp.int32),
            pltpu.VMEM((B,), jnp.int32),
            pltpu.VMEM((B,), jnp.int32),
            pltpu.VMEM((B, IN), jnp.float32),
            pltpu.VMEM((B, IN), jnp.float32),
            pltpu.VMEM_SHARED((NP, IN), jnp.float32),
            pltpu.SemaphoreType.DMA,
            pltpu.SemaphoreType.DMA,
            pltpu.SemaphoreType.DMA,
            pltpu.SemaphoreType.DMA,
            pltpu.SemaphoreType.DMA,
        ],
    )(uh, adj_pad, zeros_c)


BR = 1000


def _dinv_of(degp_blk):
    deg = degp_blk[0, :, 0:1] + degp_blk[1, :, 0:1] + 1.0
    return lax.rsqrt(deg)


def _z1u1_body(x_ref, w1_ref, degp_ref, u1_ref):
    z = jnp.dot(x_ref[...], w1_ref[...], preferred_element_type=jnp.float32)
    u = z * _dinv_of(degp_ref[...])
    u1_ref[0, :, :] = u[:, :IN]
    u1_ref[1, :, :] = u[:, IN:]


def _z1u1_call(x, W1, degp):
    return pl.pallas_call(
        _z1u1_body,
        grid=(N // BR,),
        in_specs=[
            pl.BlockSpec((BR, IN), lambda i: (i, 0)),
            pl.BlockSpec((IN, HID2), lambda i: (0, 0)),
            pl.BlockSpec((NC, BR, IN), lambda i: (0, i, 0)),
        ],
        out_specs=pl.BlockSpec((NC, BR, IN), lambda i: (0, i, 0)),
        out_shape=jax.ShapeDtypeStruct((NC, N, IN), jnp.float32),
    )(x, W1, degp)


def _hz2_body(p1_ref, u1_ref, degp_ref, b1_ref, wc_ref, u2_ref):
    dinv = _dinv_of(degp_ref[...])
    pre = jnp.concatenate(
        [p1_ref[0, :, :] + u1_ref[0, :, :], p1_ref[1, :, :] + u1_ref[1, :, :]],
        axis=1,
    )
    h = jnp.maximum(pre * dinv + b1_ref[...], 0.0)
    z2 = jnp.dot(h, wc_ref[...], preferred_element_type=jnp.float32)
    u2_ref[...] = z2 * dinv


def _hz2_call(P1, u1, degp, b1, Wc):
    return pl.pallas_call(
        _hz2_body,
        grid=(N // BR,),
        in_specs=[
            pl.BlockSpec((NC, BR, IN), lambda i: (0, i, 0)),
            pl.BlockSpec((NC, BR, IN), lambda i: (0, i, 0)),
            pl.BlockSpec((NC, BR, IN), lambda i: (0, i, 0)),
            pl.BlockSpec((1, HID2), lambda i: (0, 0)),
            pl.BlockSpec((HID2, 2 * OUT), lambda i: (0, 0)),
        ],
        out_specs=pl.BlockSpec((BR, 2 * OUT), lambda i: (i, 0)),
        out_shape=jax.ShapeDtypeStruct((N, 2 * OUT), jnp.float32),
    )(P1, u1, degp, b1, Wc)


def _out_body(p2_ref, u2_ref, degp_ref, bc_ref, o_ref):
    dinv = _dinv_of(degp_ref[...])
    pre = p2_ref[0, :, :] + p2_ref[1, :, :] + u2_ref[...]
    o_ref[...] = jnp.maximum(pre * dinv + bc_ref[...], 0.0)


def _out_call(P2, u2, degp, bc):
    return pl.pallas_call(
        _out_body,
        grid=(N // BR,),
        in_specs=[
            pl.BlockSpec((NC, BR, 2 * OUT), lambda i: (0, i, 0)),
            pl.BlockSpec((BR, 2 * OUT), lambda i: (i, 0)),
            pl.BlockSpec((NC, BR, IN), lambda i: (0, i, 0)),
            pl.BlockSpec((1, 2 * OUT), lambda i: (0, 0)),
        ],
        out_specs=pl.BlockSpec((BR, 2 * OUT), lambda i: (i, 0)),
        out_shape=jax.ShapeDtypeStruct((N, 2 * OUT), jnp.float32),
    )(P2, u2, degp, bc)


def kernel(x, adj, W1, b1, Wmu, bmu, Ws, bs):
    pad = EPAD - E
    pad_edges = jnp.stack(
        [jnp.zeros((pad,), jnp.int32), jnp.full((pad,), DUMP, jnp.int32)]
    )
    adj_pad = jnp.concatenate([adj, pad_edges], axis=1)

    zeros128 = jnp.zeros((RPT, IN), jnp.float32)
    ones_b = jnp.ones((B, IN), jnp.float32)
    Wc = jnp.concatenate([Wmu, Ws], axis=1)
    bc = jnp.concatenate([bmu, bs]).reshape(1, 2 * OUT)
    b1r = b1.reshape(1, HID2)

    degp = _deg_call(adj_pad, ones_b, zeros128)
    u1 = _z1u1_call(x, W1, degp)
    P1 = _prop_call(True, u1, adj_pad, zeros128)
    u2 = _hz2_call(P1[:, :N, :], u1, degp, b1r, Wc)
    P2 = _prop_call(False, u2, adj_pad, zeros128)
    o = _out_call(P2[:, :N, :], u2, degp, bc)
    return (o[:, :OUT], o[:, OUT:])


# speedup vs baseline: 23.0254x; 2.2991x over previous
import functools

import jax
import jax.numpy as jnp
from jax import lax
from jax.experimental import pallas as pl
from jax.experimental.pallas import tpu as pltpu
from jax.experimental.pallas import tpu_sc as plsc

N = 10000
E = 320000
IN = 128
HID2 = 256
OUT = 64

NC = 2
NS = 16
NP = 10240
RPT = NP // NS
DUMP = 10200
B = 128

EPAD = ((E + NC * NS * 2 * B - 1) // (NC * NS * 2 * B)) * (NC * NS * 2 * B)


def _sc_mesh():
    return plsc.VectorSubcoreMesh(
        core_axis_name="c", subcore_axis_name="s", num_cores=NC, num_subcores=NS
    )


KB = 16


def _deg_body(adj, ones_hbm, zeros_hbm, out, d0, d1, ones_v, acc, i0, i1, ss):
    c = lax.axis_index("c")
    s = lax.axis_index("s")
    ept = EPAD // (NS * NC)
    ebase = (s * NC + c) * ept

    pltpu.sync_copy(zeros_hbm, acc.at[pl.ds(s * RPT, RPT)])
    pltpu.sync_copy(ones_hbm, ones_v)
    plsc.subcore_barrier()

    pltpu.async_copy(adj.at[1, pl.ds(EPAD - B, B)], d1, i1)
    pltpu.make_async_copy(adj.at[1, pl.ds(EPAD - B, B)], d1, i1).wait()
    pltpu.async_copy(ones_v, acc.at[d1], ss, add=True)

    def pair(t, carry):
        a = ebase + t * 2 * B
        b = a + B
        pltpu.async_copy(adj.at[1, pl.ds(a, B)], d0, i0)
        pltpu.make_async_copy(ones_v, acc.at[d1], ss).wait()
        pltpu.async_copy(adj.at[1, pl.ds(b, B)], d1, i1)
        pltpu.make_async_copy(adj.at[1, pl.ds(a, B)], d0, i0).wait()
        pltpu.async_copy(ones_v, acc.at[d0], ss, add=True)
        pltpu.make_async_copy(adj.at[1, pl.ds(b, B)], d1, i1).wait()
        pltpu.make_async_copy(ones_v, acc.at[d0], ss).wait()
        pltpu.async_copy(ones_v, acc.at[d1], ss, add=True)
        return carry

    lax.fori_loop(0, ept // (2 * B), pair, 0)
    pltpu.make_async_copy(ones_v, acc.at[d1], ss).wait()
    plsc.subcore_barrier()
    pltpu.sync_copy(acc.at[pl.ds(s * RPT, RPT)], out.at[c, pl.ds(s * RPT, RPT)])


def _deg_call(adj_pad, ones_b, zeros_c):
    return pl.kernel(
        _deg_body,
        out_type=jax.ShapeDtypeStruct((NC, NP, IN), jnp.float32),
        mesh=_sc_mesh(),
        scratch_types=[
            pltpu.VMEM((B,), jnp.int32),
            pltpu.VMEM((B,), jnp.int32),
            pltpu.VMEM((B, IN), jnp.float32),
            pltpu.VMEM_SHARED((NP, IN), jnp.float32),
            pltpu.SemaphoreType.DMA,
            pltpu.SemaphoreType.DMA,
            pltpu.SemaphoreType.DMA,
        ],
    )(adj_pad, ones_b, zeros_c)


def _prop_body(colsplit, uh, adj, zeros_hbm, out,
               sr0, sr1, d0, d1, r0, r1, acc, i0, i1, g0, g1, ss):
    c = lax.axis_index("c")
    s = lax.axis_index("s")
    if colsplit:
        ept = EPAD // NS
        ebase = s * ept
        srcref = lambda sb: uh.at[c].at[sb]
    else:
        ept = EPAD // (NS * NC)
        ebase = (s * NC + c) * ept
        srcref = lambda sb: uh.at[sb]

    pltpu.sync_copy(zeros_hbm, acc.at[pl.ds(s * RPT, RPT)])
    plsc.subcore_barrier()

    pltpu.async_copy(adj.at[1, pl.ds(EPAD - B, B)], d1, i1)
    pltpu.make_async_copy(adj.at[1, pl.ds(EPAD - B, B)], d1, i1).wait()
    pltpu.async_copy(r1, acc.at[d1], ss, add=True)

    def pair(t, carry):
        a = ebase + t * 2 * B
        b = a + B
        pltpu.async_copy(adj.at[0, pl.ds(a, B)], sr0, i0)
        pltpu.async_copy(adj.at[1, pl.ds(a, B)], d0, i0)
        pltpu.make_async_copy(r1, acc.at[d1], ss).wait()
        pltpu.async_copy(adj.at[0, pl.ds(b, B)], sr1, i1)
        pltpu.async_copy(adj.at[1, pl.ds(b, B)], d1, i1)
        pltpu.make_async_copy(adj.at[0, pl.ds(a, B)], sr0, i0).wait()
        pltpu.make_async_copy(adj.at[1, pl.ds(a, B)], d0, i0).wait()
        pltpu.async_copy(srcref(sr0), r0, g0)
        pltpu.make_async_copy(adj.at[0, pl.ds(b, B)], sr1, i1).wait()
        pltpu.make_async_copy(adj.at[1, pl.ds(b, B)], d1, i1).wait()
        pltpu.async_copy(srcref(sr1), r1, g1)
        pltpu.make_async_copy(srcref(sr0), r0, g0).wait()
        pltpu.async_copy(r0, acc.at[d0], ss, add=True)
        pltpu.make_async_copy(srcref(sr1), r1, g1).wait()
        pltpu.make_async_copy(r0, acc.at[d0], ss).wait()
        pltpu.async_copy(r1, acc.at[d1], ss, add=True)
        return carry

    lax.fori_loop(0, ept // (2 * B), pair, 0)
    pltpu.make_async_copy(r1, acc.at[d1], ss).wait()
    plsc.subcore_barrier()
    pltpu.sync_copy(acc.at[pl.ds(s * RPT, RPT)], out.at[c, pl.ds(s * RPT, RPT)])


def _prop_call(colsplit, uh, adj_pad, zeros_c):
    return pl.kernel(
        functools.partial(_prop_body, colsplit),
        out_type=jax.ShapeDtypeStruct((NC, NP, IN), jnp.float32),
        mesh=_sc_mesh(),
        scratch_types=[
            pltpu.VMEM((B,), jnp.int32),
            pltpu.VMEM((B,), jnp.int32),
            pltpu.VMEM((B,), jnp.int32),
            pltpu.VMEM((B,), jnp.int32),
            pltpu.VMEM((B, IN), jnp.float32),
            pltpu.VMEM((B, IN), jnp.float32),
            pltpu.VMEM_SHARED((NP, IN), jnp.float32),
            pltpu.SemaphoreType.DMA,
            pltpu.SemaphoreType.DMA,
            pltpu.SemaphoreType.DMA,
            pltpu.SemaphoreType.DMA,
            pltpu.SemaphoreType.DMA,
        ],
    )(uh, adj_pad, zeros_c)


BR = 1000


def _dinv_of(degp_blk):
    deg = degp_blk[0, :, 0:1] + degp_blk[1, :, 0:1] + 1.0
    return lax.rsqrt(deg)


def _z1u1_body(x_ref, w1_ref, degp_ref, u1_ref):
    z = jnp.dot(x_ref[...], w1_ref[...], preferred_element_type=jnp.float32)
    u = z * _dinv_of(degp_ref[...])
    u1_ref[0, :, :] = u[:, :IN]
    u1_ref[1, :, :] = u[:, IN:]


def _z1u1_call(x, W1, degp):
    return pl.pallas_call(
        _z1u1_body,
        grid=(N // BR,),
        in_specs=[
            pl.BlockSpec((BR, IN), lambda i: (i, 0)),
            pl.BlockSpec((IN, HID2), lambda i: (0, 0)),
            pl.BlockSpec((NC, BR, IN), lambda i: (0, i, 0)),
        ],
        out_specs=pl.BlockSpec((NC, BR, IN), lambda i: (0, i, 0)),
        out_shape=jax.ShapeDtypeStruct((NC, N, IN), jnp.float32),
    )(x, W1, degp)


def _hz2_body(p1_ref, u1_ref, degp_ref, b1_ref, wc_ref, u2_ref):
    dinv = _dinv_of(degp_ref[...])
    pre = jnp.concatenate(
        [p1_ref[0, :, :] + u1_ref[0, :, :], p1_ref[1, :, :] + u1_ref[1, :, :]],
        axis=1,
    )
    h = jnp.maximum(pre * dinv + b1_ref[...], 0.0)
    z2 = jnp.dot(h, wc_ref[...], preferred_element_type=jnp.float32)
    u2_ref[...] = z2 * dinv


def _hz2_call(P1, u1, degp, b1, Wc):
    return pl.pallas_call(
        _hz2_body,
        grid=(N // BR,),
        in_specs=[
            pl.BlockSpec((NC, BR, IN), lambda i: (0, i, 0)),
            pl.BlockSpec((NC, BR, IN), lambda i: (0, i, 0)),
            pl.BlockSpec((NC, BR, IN), lambda i: (0, i, 0)),
            pl.BlockSpec((1, HID2), lambda i: (0, 0)),
            pl.BlockSpec((HID2, 2 * OUT), lambda i: (0, 0)),
        ],
        out_specs=pl.BlockSpec((BR, 2 * OUT), lambda i: (i, 0)),
        out_shape=jax.ShapeDtypeStruct((N, 2 * OUT), jnp.float32),
    )(P1, u1, degp, b1, Wc)


def _out_body(p2_ref, u2_ref, degp_ref, bc_ref, o_ref):
    dinv = _dinv_of(degp_ref[...])
    pre = p2_ref[0, :, :] + p2_ref[1, :, :] + u2_ref[...]
    o_ref[...] = jnp.maximum(pre * dinv + bc_ref[...], 0.0)


def _out_call(P2, u2, degp, bc):
    return pl.pallas_call(
        _out_body,
        grid=(N // BR,),
        in_specs=[
            pl.BlockSpec((NC, BR, 2 * OUT), lambda i: (0, i, 0)),
            pl.BlockSpec((BR, 2 * OUT), lambda i: (i, 0)),
            pl.BlockSpec((NC, BR, IN), lambda i: (0, i, 0)),
            pl.BlockSpec((1, 2 * OUT), lambda i: (0, 0)),
        ],
        out_specs=pl.BlockSpec((BR, 2 * OUT), lambda i: (i, 0)),
        out_shape=jax.ShapeDtypeStruct((N, 2 * OUT), jnp.float32),
    )(P2, u2, degp, bc)


def kernel(x, adj, W1, b1, Wmu, bmu, Ws, bs):
    pad = EPAD - E
    pidx = jnp.arange(pad, dtype=jnp.int32)
    pad_edges = jnp.stack([pidx % N, N + pidx % (NP - N)])
    adj_pad = jnp.concatenate([adj, pad_edges], axis=1)

    zeros128 = jnp.zeros((RPT, IN), jnp.float32)
    ones_b = jnp.ones((B, IN), jnp.float32)
    Wc = jnp.concatenate([Wmu, Ws], axis=1)
    bc = jnp.concatenate([bmu, bs]).reshape(1, 2 * OUT)
    b1r = b1.reshape(1, HID2)

    degp = _deg_call(adj_pad, ones_b, zeros128)
    u1 = _z1u1_call(x, W1, degp)
    P1 = _prop_call(True, u1, adj_pad, zeros128)
    u2 = _hz2_call(P1[:, :N, :], u1, degp, b1r, Wc)
    P2 = _prop_call(False, u2, adj_pad, zeros128)
    o = _out_call(P2[:, :N, :], u2, degp, bc)
    return (o[:, :OUT], o[:, OUT:])


# speedup vs baseline: 29.3905x; 1.2764x over previous
import jax
import jax.numpy as jnp
from jax import lax
from jax.experimental import pallas as pl
from jax.experimental.pallas import tpu as pltpu
from jax.experimental.pallas import tpu_sc as plsc

N = 10000
E = 320000
IN = 128
HID2 = 256
OUT = 64

NC = 2
NS = 16
NP = 10240
RPT = NP // NS
DUMP = 10200
B = 128

EPAD = ((E + NC * NS * 2 * B - 1) // (NC * NS * 2 * B)) * (NC * NS * 2 * B)


def _sc_mesh():
    return plsc.VectorSubcoreMesh(
        core_axis_name="c", subcore_axis_name="s", num_cores=NC, num_subcores=NS
    )


KB = 16


def _deg_body(adj, ones_hbm, zeros_hbm, out, d0, d1, ones_v, acc, i0, i1, ss):
    c = lax.axis_index("c")
    s = lax.axis_index("s")
    ept = EPAD // (NS * NC)
    ebase = (s * NC + c) * ept

    pltpu.sync_copy(zeros_hbm, acc.at[pl.ds(s * RPT, RPT)])
    pltpu.sync_copy(ones_hbm, ones_v)
    plsc.subcore_barrier()

    pltpu.async_copy(adj.at[1, pl.ds(EPAD - B, B)], d1, i1)
    pltpu.make_async_copy(adj.at[1, pl.ds(EPAD - B, B)], d1, i1).wait()
    pltpu.async_copy(ones_v, acc.at[d1], ss, add=True)

    def pair(t, carry):
        a = ebase + t * 2 * B
        b = a + B
        pltpu.async_copy(adj.at[1, pl.ds(a, B)], d0, i0)
        pltpu.make_async_copy(ones_v, acc.at[d1], ss).wait()
        pltpu.async_copy(adj.at[1, pl.ds(b, B)], d1, i1)
        pltpu.make_async_copy(adj.at[1, pl.ds(a, B)], d0, i0).wait()
        pltpu.async_copy(ones_v, acc.at[d0], ss, add=True)
        pltpu.make_async_copy(adj.at[1, pl.ds(b, B)], d1, i1).wait()
        pltpu.make_async_copy(ones_v, acc.at[d0], ss).wait()
        pltpu.async_copy(ones_v, acc.at[d1], ss, add=True)
        return carry

    lax.fori_loop(0, ept // (2 * B), pair, 0)
    pltpu.make_async_copy(ones_v, acc.at[d1], ss).wait()
    plsc.subcore_barrier()
    pltpu.sync_copy(acc.at[pl.ds(s * RPT, RPT)], out.at[c, pl.ds(s * RPT, RPT)])


def _deg_call(adj_pad, ones_b, zeros_c):
    return pl.kernel(
        _deg_body,
        out_type=jax.ShapeDtypeStruct((NC, NP, IN), jnp.float32),
        mesh=_sc_mesh(),
        scratch_types=[
            pltpu.VMEM((B,), jnp.int32),
            pltpu.VMEM((B,), jnp.int32),
            pltpu.VMEM((B, IN), jnp.float32),
            pltpu.VMEM_SHARED((NP, IN), jnp.float32),
            pltpu.SemaphoreType.DMA,
            pltpu.SemaphoreType.DMA,
            pltpu.SemaphoreType.DMA,
        ],
    )(adj_pad, ones_b, zeros_c)


def _prop_body(uh, adj, zeros_hbm, out,
               sr0, sr1, d0, d1, r0, r1, acc, i0, i1, g0, g1, ss):
    c = lax.axis_index("c")
    s = lax.axis_index("s")
    ept = EPAD // (NS * NC)
    ebase = (s * NC + c) * ept
    srcref = lambda sb: uh.at[sb]

    pltpu.sync_copy(zeros_hbm, acc.at[pl.ds(s * RPT, RPT)])
    plsc.subcore_barrier()

    pltpu.async_copy(adj.at[1, pl.ds(EPAD - B, B)], d1, i1)
    pltpu.make_async_copy(adj.at[1, pl.ds(EPAD - B, B)], d1, i1).wait()
    pltpu.async_copy(r1, acc.at[d1], ss, add=True)

    def pair(t, carry):
        a = ebase + t * 2 * B
        b = a + B
        pltpu.async_copy(adj.at[0, pl.ds(a, B)], sr0, i0)
        pltpu.async_copy(adj.at[1, pl.ds(a, B)], d0, i0)
        pltpu.make_async_copy(r1, acc.at[d1], ss).wait()
        pltpu.async_copy(adj.at[0, pl.ds(b, B)], sr1, i1)
        pltpu.async_copy(adj.at[1, pl.ds(b, B)], d1, i1)
        pltpu.make_async_copy(adj.at[0, pl.ds(a, B)], sr0, i0).wait()
        pltpu.make_async_copy(adj.at[1, pl.ds(a, B)], d0, i0).wait()
        pltpu.async_copy(srcref(sr0), r0, g0)
        pltpu.make_async_copy(adj.at[0, pl.ds(b, B)], sr1, i1).wait()
        pltpu.make_async_copy(adj.at[1, pl.ds(b, B)], d1, i1).wait()
        pltpu.async_copy(srcref(sr1), r1, g1)
        pltpu.make_async_copy(srcref(sr0), r0, g0).wait()
        pltpu.async_copy(r0, acc.at[d0], ss, add=True)
        pltpu.make_async_copy(srcref(sr1), r1, g1).wait()
        pltpu.make_async_copy(r0, acc.at[d0], ss).wait()
        pltpu.async_copy(r1, acc.at[d1], ss, add=True)
        return carry

    lax.fori_loop(0, ept // (2 * B), pair, 0)
    pltpu.make_async_copy(r1, acc.at[d1], ss).wait()
    plsc.subcore_barrier()
    pltpu.sync_copy(acc.at[pl.ds(s * RPT, RPT)], out.at[c, pl.ds(s * RPT, RPT)])


def _prop_call(uh, adj_pad, zeros_c):
    return pl.kernel(
        _prop_body,
        out_type=jax.ShapeDtypeStruct((NC, NP, IN), jnp.float32),
        mesh=_sc_mesh(),
        scratch_types=[
            pltpu.VMEM((B,), jnp.int32),
            pltpu.VMEM((B,), jnp.int32),
            pltpu.VMEM((B,), jnp.int32),
            pltpu.VMEM((B,), jnp.int32),
            pltpu.VMEM((B, IN), jnp.float32),
            pltpu.VMEM((B, IN), jnp.float32),
            pltpu.VMEM_SHARED((NP, IN), jnp.float32),
            pltpu.SemaphoreType.DMA,
            pltpu.SemaphoreType.DMA,
            pltpu.SemaphoreType.DMA,
            pltpu.SemaphoreType.DMA,
            pltpu.SemaphoreType.DMA,
        ],
    )(uh, adj_pad, zeros_c)


BR = 1000


def _ux_body(x_ref, degp_ref, ux_ref, dinv_ref):
    deg = degp_ref[0, :, 0:1] + degp_ref[1, :, 0:1] + 1.0
    dinv = lax.rsqrt(deg)
    ux_ref[...] = x_ref[...] * dinv
    dinv_ref[...] = dinv


def _ux_call(x, degp):
    return pl.pallas_call(
        _ux_body,
        grid=(N // BR,),
        in_specs=[
            pl.BlockSpec((BR, IN), lambda i: (i, 0)),
            pl.BlockSpec((NC, BR, IN), lambda i: (0, i, 0)),
        ],
        out_specs=[
            pl.BlockSpec((BR, IN), lambda i: (i, 0)),
            pl.BlockSpec((BR, 1), lambda i: (i, 0)),
        ],
        out_shape=[
            jax.ShapeDtypeStruct((N, IN), jnp.float32),
            jax.ShapeDtypeStruct((N, 1), jnp.float32),
        ],
    )(x, degp)


def _hz2_body(px_ref, ux_ref, dinv_ref, b1_ref, w1_ref, wc_ref, u2_ref):
    dinv = dinv_ref[...]
    ax = (px_ref[0, :, :] + px_ref[1, :, :] + ux_ref[...]) * dinv
    h = jnp.maximum(
        jnp.dot(ax, w1_ref[...], preferred_element_type=jnp.float32)
        + b1_ref[...],
        0.0,
    )
    z2 = jnp.dot(h, wc_ref[...], preferred_element_type=jnp.float32)
    u2_ref[...] = z2 * dinv


def _hz2_call(Px, ux, dinv, b1, W1, Wc):
    return pl.pallas_call(
        _hz2_body,
        grid=(N // BR,),
        in_specs=[
            pl.BlockSpec((NC, BR, IN), lambda i: (0, i, 0)),
            pl.BlockSpec((BR, IN), lambda i: (i, 0)),
            pl.BlockSpec((BR, 1), lambda i: (i, 0)),
            pl.BlockSpec((1, HID2), lambda i: (0, 0)),
            pl.BlockSpec((IN, HID2), lambda i: (0, 0)),
            pl.BlockSpec((HID2, 2 * OUT), lambda i: (0, 0)),
        ],
        out_specs=pl.BlockSpec((BR, 2 * OUT), lambda i: (i, 0)),
        out_shape=jax.ShapeDtypeStruct((N, 2 * OUT), jnp.float32),
    )(Px, ux, dinv, b1, W1, Wc)


def _out_body(p2_ref, u2_ref, dinv_ref, bc_ref, o_ref):
    pre = p2_ref[0, :, :] + p2_ref[1, :, :] + u2_ref[...]
    o_ref[...] = jnp.maximum(pre * dinv_ref[...] + bc_ref[...], 0.0)


def _out_call(P2, u2, dinv, bc):
    return pl.pallas_call(
        _out_body,
        grid=(N // BR,),
        in_specs=[
            pl.BlockSpec((NC, BR, 2 * OUT), lambda i: (0, i, 0)),
            pl.BlockSpec((BR, 2 * OUT), lambda i: (i, 0)),
            pl.BlockSpec((BR, 1), lambda i: (i, 0)),
            pl.BlockSpec((1, 2 * OUT), lambda i: (0, 0)),
        ],
        out_specs=pl.BlockSpec((BR, 2 * OUT), lambda i: (i, 0)),
        out_shape=jax.ShapeDtypeStruct((N, 2 * OUT), jnp.float32),
    )(P2, u2, dinv, bc)


def kernel(x, adj, W1, b1, Wmu, bmu, Ws, bs):
    pad = EPAD - E
    pidx = jnp.arange(pad, dtype=jnp.int32)
    pad_edges = jnp.stack([pidx % N, N + pidx % (NP - N)])
    adj_pad = jnp.concatenate([adj, pad_edges], axis=1)

    zeros128 = jnp.zeros((RPT, IN), jnp.float32)
    ones_b = jnp.ones((B, IN), jnp.float32)
    Wc = jnp.concatenate([Wmu, Ws], axis=1)
    bc = jnp.concatenate([bmu, bs]).reshape(1, 2 * OUT)
    b1r = b1.reshape(1, HID2)

    degp = _deg_call(adj_pad, ones_b, zeros128)
    ux, dinv = _ux_call(x, degp)
    Px = _prop_call(ux, adj_pad, zeros128)
    u2 = _hz2_call(Px[:, :N, :], ux, dinv, b1r, W1, Wc)
    P2 = _prop_call(u2, adj_pad, zeros128)
    o = _out_call(P2[:, :N, :], u2, dinv, bc)
    return (o[:, :OUT], o[:, OUT:])
